# baseline (speedup 1.0000x reference)
import functools

import jax
import jax.numpy as jnp
from jax import lax
from jax.experimental import pallas as pl
from jax.experimental.pallas import tpu as pltpu
from jax.experimental.pallas import tpu_sc as plsc

N = 10000
E = 320000
D = 128
H = 128
HH = H // 2
G = 64

N_PAD = 10240
CHUNK = 125
IDX_ROWS = E // CHUNK
DEG_ROWS = IDX_ROWS // 32
AGG_ROWS = IDX_ROWS // 16
E_TILE = E // 32
TILE_SLICE = N_PAD // 16
NBUF = 4
BLK = 1000
NBLK = N // BLK

_mesh = plsc.VectorSubcoreMesh(core_axis_name="c", subcore_axis_name="s")
_sc_params = pltpu.CompilerParams(use_tc_tiling_on_sc=False)
_sc_gather_params = pltpu.CompilerParams(use_tc_tiling_on_sc=False,
                                         needs_layout_passes=False)


def _sc_degree(dst2d, ones_c, zeros16):
    @functools.partial(
        pl.kernel,
        mesh=_mesh,
        out_type=jax.ShapeDtypeStruct((2, N_PAD, 16), jnp.float32),
        compiler_params=_sc_params,
        scratch_types=[
            pltpu.VMEM((DEG_ROWS, CHUNK), jnp.int32),
            pltpu.VMEM((CHUNK, 16), jnp.float32),
            pltpu.VMEM_SHARED((N_PAD, 16), jnp.float32),
        ],
    )
    def k(dst_hbm, ones_hbm, z_hbm, out_hbm, didx, ones_v, acc):
        cid = lax.axis_index("c")
        sid = lax.axis_index("s")
        wrow = (cid * 16 + sid) * DEG_ROWS
        pltpu.sync_copy(dst_hbm.at[pl.ds(wrow, DEG_ROWS)], didx)
        pltpu.sync_copy(ones_hbm, ones_v)
        pltpu.sync_copy(z_hbm, acc.at[pl.ds(sid * TILE_SLICE, TILE_SLICE)])
        plsc.subcore_barrier()

        @pl.loop(0, DEG_ROWS)
        def _(r):
            pltpu.sync_copy(ones_v, acc.at[didx.at[r]], add=True)

        plsc.subcore_barrier()
        pltpu.sync_copy(
            acc.at[pl.ds(sid * TILE_SLICE, TILE_SLICE)],
            out_hbm.at[cid, pl.ds(sid * TILE_SLICE, TILE_SLICE)],
        )

    return k(dst2d, ones_c, zeros16)


def _sc_norm(dinv_flat, srcN, dstN):
    @functools.partial(
        pl.kernel,
        mesh=_mesh,
        out_type=jax.ShapeDtypeStruct((32, E_TILE), jnp.float32),
        compiler_params=_sc_gather_params,
        scratch_types=[
            pltpu.VMEM((N_PAD,), jnp.float32),
            pltpu.VMEM((E_TILE,), jnp.int32),
            pltpu.VMEM((E_TILE,), jnp.int32),
            pltpu.VMEM((E_TILE,), jnp.float32),
        ],
    )
    def k(dinv_hbm, src_hbm, dst_hbm, out_hbm, df, sflat, dflat, nflat):
        cid = lax.axis_index("c")
        sid = lax.axis_index("s")
        tid = cid * 16 + sid
        pltpu.sync_copy(dinv_hbm, df)
        pltpu.sync_copy(src_hbm.at[tid], sflat)
        pltpu.sync_copy(dst_hbm.at[tid], dflat)

        @pl.loop(0, E_TILE // 16)
        def _(v):
            sl = pl.ds(v * 16, 16)
            ns = plsc.load_gather(df, [sflat[sl]])
            nd = plsc.load_gather(df, [dflat[sl]])
            nflat[sl] = ns * nd

        pltpu.sync_copy(nflat, out_hbm.at[tid])

    return k(dinv_flat, srcN, dstN)


def _sc_aggregate(xw2, src2d, dst2d, norm16, zeros64):
    @functools.partial(
        pl.kernel,
        mesh=_mesh,
        out_type=jax.ShapeDtypeStruct((2, N_PAD, HH), jnp.float32),
        compiler_params=_sc_params,
        scratch_types=(
            [pltpu.VMEM((AGG_ROWS, CHUNK), jnp.int32),
             pltpu.VMEM((AGG_ROWS, CHUNK), jnp.int32),
             pltpu.VMEM_SHARED((N_PAD, HH), jnp.float32)]
            + [pltpu.VMEM((CHUNK, HH), jnp.float32)] * NBUF
            + [pltpu.VMEM((CHUNK, 16), jnp.float32)] * NBUF
            + [pltpu.SemaphoreType.DMA] * (3 * NBUF)
        ),
    )
    def k(y_hbm, src_hbm, dst_hbm, nrm_hbm, z_hbm, out_hbm,
          sidx, didx, acc, *rest):
        bufs = rest[:NBUF]
        nbufs = rest[NBUF:2 * NBUF]
        gsems = rest[2 * NBUF:3 * NBUF]
        ssems = rest[3 * NBUF:4 * NBUF]
        nsems = rest[4 * NBUF:]
        cid = lax.axis_index("c")
        sid = lax.axis_index("s")
        wrow = sid * AGG_ROWS
        yc = y_hbm.at[cid]
        pltpu.sync_copy(src_hbm.at[pl.ds(wrow, AGG_ROWS)], sidx)
        pltpu.sync_copy(dst_hbm.at[pl.ds(wrow, AGG_ROWS)], didx)
        pltpu.sync_copy(z_hbm, acc.at[pl.ds(sid * TILE_SLICE, TILE_SLICE)])
        plsc.subcore_barrier()

        def scale_rows(k_):
            @pl.loop(0, CHUNK)
            def _(row):
                nv = nbufs[k_][row, :]
                for c in range(HH // 16):
                    sl = pl.ds(c * 16, 16)
                    bufs[k_][row, sl] = bufs[k_][row, sl] * nv

        for k2 in range(NBUF):
            pltpu.async_copy(yc.at[sidx.at[k2]], bufs[k2], gsems[k2])
            pltpu.async_copy(nrm_hbm.at[wrow + k2], nbufs[k2], nsems[k2])

        @pl.loop(0, AGG_ROWS // NBUF)
        def _(j):
            r = j * NBUF
            scat = []
            for k2 in range(NBUF):
                pltpu.make_async_copy(
                    yc.at[sidx.at[r + k2]], bufs[k2], gsems[k2]).wait()
                pltpu.make_async_copy(
                    nrm_hbm.at[wrow + r + k2], nbufs[k2], nsems[k2]).wait()
                scale_rows(k2)
                scat.append(pltpu.async_copy(
                    bufs[k2], acc.at[didx.at[r + k2]], ssems[k2], add=True))
            for k2 in range(NBUF):
                scat[k2].wait()

                @pl.when(r + k2 + NBUF < AGG_ROWS)
                def _():
                    pltpu.async_copy(
                        yc.at[sidx.at[r + k2 + NBUF]], bufs[k2], gsems[k2])
                    pltpu.async_copy(
                        nrm_hbm.at[wrow + r + k2 + NBUF], nbufs[k2],
                        nsems[k2])

        plsc.subcore_barrier()
        pltpu.sync_copy(
            acc.at[pl.ds(sid * TILE_SLICE, TILE_SLICE)],
            out_hbm.at[cid, pl.ds(sid * TILE_SLICE, TILE_SLICE)],
        )

    return k(xw2, src2d, dst2d, norm16, zeros64)


def _dot(a, b):
    return lax.dot_general(a, b, (((1,), (0,)), ((), ())),
                           preferred_element_type=jnp.float32)


def _split(o_ref, y):
    o_ref[0] = y[:, :HH]
    o_ref[1] = y[:, HH:]


def _deg(d_ref):
    return d_ref[0, :, :1] + d_ref[1, :, :1] + 1.0


def _tc_matmul0(x, w):
    def body(x_ref, w_ref, o_ref):
        _split(o_ref, _dot(x_ref[...], w_ref[...]))

    return pl.pallas_call(
        body,
        grid=(NBLK,),
        in_specs=[
            pl.BlockSpec((BLK, D), lambda i: (i, 0)),
            pl.BlockSpec((D, H), lambda i: (0, 0)),
        ],
        out_specs=pl.BlockSpec((2, BLK, HH), lambda i: (0, i, 0)),
        out_shape=jax.ShapeDtypeStruct((2, N, HH), jnp.float32),
    )(x, w)


def _tc_norm16(norm_col):
    EB = 10000

    def body(n_ref, o_ref):
        o_ref[...] = jnp.broadcast_to(n_ref[...], (EB, 16))

    return pl.pallas_call(
        body,
        grid=(E // EB,),
        in_specs=[pl.BlockSpec((EB, 1), lambda i: (i, 0))],
        out_specs=pl.BlockSpec((EB, 16), lambda i: (i, 0)),
        out_shape=jax.ShapeDtypeStruct((E, 16), jnp.float32),
    )(norm_col)


def _tc_dinv(degp):
    def body(d_ref, o_ref):
        o_ref[...] = lax.rsqrt(_deg(d_ref))

    NPB = N_PAD // 8

    return pl.pallas_call(
        body,
        grid=(8,),
        in_specs=[pl.BlockSpec((2, NPB, 16), lambda i: (0, i, 0))],
        out_specs=pl.BlockSpec((NPB, 1), lambda i: (i, 0)),
        out_shape=jax.ShapeDtypeStruct((N_PAD, 1), jnp.float32),
    )(degp)


def _h_block(p_ref, y_ref, d_ref, b_ref):
    dinv = lax.rsqrt(_deg(d_ref))
    nself = dinv * dinv
    agg = jnp.concatenate([p_ref[0], p_ref[1]], axis=1)
    xw = jnp.concatenate([y_ref[0], y_ref[1]], axis=1)
    return jnp.maximum(agg + nself * xw + b_ref[...], 0.0)


def _tc_layer(parts, xw2, degp, b, w_next):
    def body(p_ref, y_ref, d_ref, b_ref, w_ref, o_ref):
        h = _h_block(p_ref, y_ref, d_ref, b_ref)
        _split(o_ref, _dot(h, w_ref[...]))

    return pl.pallas_call(
        body,
        grid=(NBLK,),
        in_specs=[
            pl.BlockSpec((2, BLK, HH), lambda i: (0, i, 0)),
            pl.BlockSpec((2, BLK, HH), lambda i: (0, i, 0)),
            pl.BlockSpec((2, BLK, 16), lambda i: (0, i, 0)),
            pl.BlockSpec((1, H), lambda i: (0, 0)),
            pl.BlockSpec((H, H), lambda i: (0, 0)),
        ],
        out_specs=pl.BlockSpec((2, BLK, HH), lambda i: (0, i, 0)),
        out_shape=jax.ShapeDtypeStruct((2, N, HH), jnp.float32),
    )(parts, xw2, degp, b, w_next)


def _tc_final(parts, xw2, degp, b, batch2d, w0, b0, w1, b1):
    def body(p_ref, y_ref, d_ref, b_ref, bat_ref, w0_ref, b0_ref, w1_ref,
             b1_ref, o_ref, acc_ref):
        i = pl.program_id(0)

        @pl.when(i == 0)
        def _():
            acc_ref[...] = jnp.full((G, H), -jnp.inf, jnp.float32)

        h = _h_block(p_ref, y_ref, d_ref, b_ref)
        bat = bat_ref[...]
        g_lo = bat_ref[0, 0]
        g_hi = bat_ref[BLK - 1, 0]
        seg_col = lax.broadcasted_iota(jnp.int32, (G, 1), 0)

        def seg_body(g, m):
            sel = jnp.where(bat == g, h, -jnp.inf)
            row = jnp.max(sel, axis=0, keepdims=True)
            return jnp.maximum(m, jnp.where(seg_col == g, row, -jnp.inf))

        acc_ref[...] = lax.fori_loop(g_lo, g_hi + 1, seg_body, acc_ref[...])

        @pl.when(i == NBLK - 1)
        def _():
            g1 = _dot(acc_ref[...], w0_ref[...]) + b0_ref[...]
            o_ref[...] = _dot(g1, w1_ref[...]) + b1_ref[...]

    return pl.pallas_call(
        body,
        grid=(NBLK,),
        in_specs=[
            pl.BlockSpec((2, BLK, HH), lambda i: (0, i, 0)),
            pl.BlockSpec((2, BLK, HH), lambda i: (0, i, 0)),
            pl.BlockSpec((2, BLK, 16), lambda i: (0, i, 0)),
            pl.BlockSpec((1, H), lambda i: (0, 0)),
            pl.BlockSpec((BLK, 1), lambda i: (i, 0)),
            pl.BlockSpec((H, H), lambda i: (0, 0)),
            pl.BlockSpec((1, H), lambda i: (0, 0)),
            pl.BlockSpec((H, 1), lambda i: (0, 0)),
            pl.BlockSpec((1, 1), lambda i: (0, 0)),
        ],
        out_specs=pl.BlockSpec((G, 1), lambda i: (0, 0)),
        out_shape=jax.ShapeDtypeStruct((G, 1), jnp.float32),
        scratch_shapes=[pltpu.VMEM((G, H), jnp.float32)],
    )(parts, xw2, degp, b, batch2d, w0, b0, w1, b1)


def kernel(x_p, edge_index_p, x_p_batch, W_conv0, b_conv0, W_conv1, b_conv1,
           W_conv2, b_conv2, W_lin0, b_lin0, W_lin1, b_lin1):
    src2d = edge_index_p[0].reshape(IDX_ROWS, CHUNK)
    dst2d = edge_index_p[1].reshape(IDX_ROWS, CHUNK)
    srcN = edge_index_p[0].reshape(32, E_TILE)
    dstN = edge_index_p[1].reshape(32, E_TILE)
    batch2d = x_p_batch.reshape(N, 1)
    ones_c = jnp.ones((CHUNK, 16), jnp.float32)
    zeros16 = jnp.zeros((TILE_SLICE, 16), jnp.float32)
    zeros64 = jnp.zeros((TILE_SLICE, HH), jnp.float32)
    b0 = b_conv0.reshape(1, H)
    b1 = b_conv1.reshape(1, H)
    b2 = b_conv2.reshape(1, H)
    bl0 = b_lin0.reshape(1, H)
    bl1 = b_lin1.reshape(1, 1)

    degp = _sc_degree(dst2d, ones_c, zeros16)
    dinv_flat = _tc_dinv(degp).reshape(N_PAD)
    norm_col = _sc_norm(dinv_flat, srcN, dstN).reshape(E, 1)
    norm16 = _tc_norm16(norm_col).reshape(IDX_ROWS, CHUNK, 16)
    xw0 = _tc_matmul0(x_p, W_conv0)
    p0 = _sc_aggregate(xw0, src2d, dst2d, norm16, zeros64)
    xw1 = _tc_layer(p0, xw0, degp, b0, W_conv1)
    p1 = _sc_aggregate(xw1, src2d, dst2d, norm16, zeros64)
    xw2 = _tc_layer(p1, xw1, degp, b1, W_conv2)
    p2 = _sc_aggregate(xw2, src2d, dst2d, norm16, zeros64)
    return _tc_final(p2, xw2, degp, b2, batch2d, W_lin0, bl0, W_lin1, bl1)
